# baseline (speedup 1.0000x reference)
import functools

import jax
import jax.numpy as jnp
from jax import lax
from jax.experimental import pallas as pl
from jax.experimental.pallas import tpu as pltpu
from jax.experimental.pallas import tpu_sc as plsc

NC = 2
NS = 16
NW = NC * NS
LANE = 16
EG = 128
NBUF = 4
DEFER = 2


def _make_deg_kernel(n_pad: int, e_pad: int):
    per_w = e_pad // NW
    mesh = plsc.VectorSubcoreMesh(
        core_axis_name="c", subcore_axis_name="s",
        num_cores=NC, num_subcores=NS)

    @functools.partial(
        pl.kernel, mesh=mesh,
        compiler_params=pltpu.CompilerParams(needs_layout_passes=False),
        out_type=jax.ShapeDtypeStruct((NW, n_pad), jnp.float32),
        scratch_types=[
            pltpu.VMEM((per_w,), jnp.int32),
            pltpu.VMEM((n_pad,), jnp.float32),
        ],
    )
    def deg_kernel(dst_hbm, out_hbm, dst_v, hist_v):
        cid = lax.axis_index("c")
        sid = lax.axis_index("s")
        wid = cid * NS + sid
        pltpu.sync_copy(dst_hbm.at[pl.ds(wid * per_w, per_w)], dst_v)
        zeros16 = jnp.zeros((LANE,), jnp.float32)
        ones16 = jnp.ones((LANE,), jnp.float32)

        def zero_body(i, carry):
            hist_v[pl.ds(i * LANE, LANE)] = zeros16
            return carry

        lax.fori_loop(0, n_pad // LANE, zero_body, 0)

        def acc_body(t, carry):
            idx = dst_v[pl.ds(t * LANE, LANE)]
            plsc.addupdate_scatter(hist_v, [idx], ones16)
            return carry

        lax.fori_loop(0, per_w // LANE, acc_body, 0)
        pltpu.sync_copy(hist_v, out_hbm.at[wid])

    return deg_kernel


def _make_edge_kernel(n_pad: int, groups_total: int, hd: int):
    gps = groups_total // NS
    zrows = n_pad // NS
    mesh = plsc.VectorSubcoreMesh(
        core_axis_name="c", subcore_axis_name="s",
        num_cores=NC, num_subcores=NS)

    @functools.partial(
        pl.kernel, mesh=mesh,
        compiler_params=pltpu.CompilerParams(
            needs_layout_passes=False, use_tc_tiling_on_sc=False),
        out_type=jax.ShapeDtypeStruct((NC, n_pad, hd), jnp.float32),
        scratch_types=[
            pltpu.VMEM((gps, EG), jnp.int32),
            pltpu.VMEM((gps, EG), jnp.int32),
        ] + [pltpu.VMEM((EG, hd), jnp.float32) for _ in range(NBUF)] + [
            pltpu.VMEM_SHARED((n_pad, hd), jnp.float32),
        ] + [pltpu.SemaphoreType.DMA for _ in range(2 * NBUF)],
    )
    def edge_kernel(g_hbm, src_hbm, dst_hbm, z_hbm, out_hbm,
                    src_v, dst_v, *rest):
        rows = rest[:NBUF]
        acc = rest[NBUF]
        gsems = rest[NBUF + 1:NBUF + 1 + NBUF]
        ssems = rest[NBUF + 1 + NBUF:]
        cid = lax.axis_index("c")
        sid = lax.axis_index("s")
        table = g_hbm.at[cid]
        pltpu.sync_copy(z_hbm, acc.at[pl.ds(sid * zrows, zrows)])
        pltpu.sync_copy(src_hbm.at[pl.ds(sid * gps, gps)], src_v)
        pltpu.sync_copy(dst_hbm.at[pl.ds(sid * gps, gps)], dst_v)
        for b in range(NBUF):
            pltpu.async_copy(table.at[src_v.at[b]], rows[b], gsems[b])
        plsc.subcore_barrier()

        def chunk_body(jc, carry):
            j0 = jc * NBUF
            for b in range(NBUF):
                j = j0 + b
                pltpu.make_async_copy(
                    table.at[src_v.at[j]], rows[b], gsems[b]).wait()
                pltpu.async_copy(rows[b], acc.at[dst_v.at[j]], ssems[b],
                                 add=True)
                bq = (b - DEFER) % NBUF
                nxt = j + NBUF - DEFER

                @pl.when(jnp.logical_and(j >= DEFER, nxt < gps))
                def _refill():
                    pltpu.make_async_copy(
                        rows[bq], acc.at[dst_v.at[0]], ssems[bq]).wait()
                    pltpu.async_copy(
                        table.at[src_v.at[nxt]], rows[bq], gsems[bq])
            return carry

        lax.fori_loop(0, gps // NBUF, chunk_body, 0)
        for b in range(NBUF):
            pltpu.make_async_copy(
                rows[b], acc.at[dst_v.at[0]], ssems[b]).wait()
        plsc.subcore_barrier()
        pltpu.sync_copy(acc.at[pl.ds(sid * zrows, zrows)],
                        out_hbm.at[cid, pl.ds(sid * zrows, zrows)])

    return edge_kernel


_BN = 1024


def _split(h):
    hd = h.shape[-1] // 2
    return jnp.stack([h[:, :hd], h[:, hd:]], axis=0)


def _tc_first(x, w1, degs):
    n_pad, d = x.shape

    def body(x_ref, w_ref, deg_ref, g_ref, dinv_ref):
        deg = jnp.sum(deg_ref[...], axis=0) + 1.0
        dinv = lax.rsqrt(deg)[:, None]
        h = jnp.dot(x_ref[...], w_ref[...],
                    preferred_element_type=jnp.float32)
        g_ref[...] = _split(h * dinv)
        dinv_ref[...] = dinv

    return pl.pallas_call(
        body,
        grid=(n_pad // _BN,),
        in_specs=[
            pl.BlockSpec((_BN, d), lambda i: (i, 0)),
            pl.BlockSpec((d, d), lambda i: (0, 0)),
            pl.BlockSpec((NW, _BN), lambda i: (0, i)),
        ],
        out_specs=[
            pl.BlockSpec((NC, _BN, d // 2), lambda i: (0, i, 0)),
            pl.BlockSpec((_BN, 1), lambda i: (i, 0)),
        ],
        out_shape=[
            jax.ShapeDtypeStruct((NC, n_pad, d // 2), jnp.float32),
            jax.ShapeDtypeStruct((n_pad, 1), jnp.float32),
        ],
    )(x, w1, degs)


def _tc_mid(p, g1, dinv, b1, w2):
    _, n_pad, hd = g1.shape
    d = 2 * hd

    def body(p_ref, g_ref, dinv_ref, b_ref, w_ref, out_ref):
        s = jnp.concatenate([p_ref[0] + g_ref[0], p_ref[1] + g_ref[1]],
                            axis=-1)
        h = jnp.maximum(dinv_ref[...] * s + b_ref[...][None, :], 0.0)
        g2 = jnp.dot(h, w_ref[...],
                     preferred_element_type=jnp.float32) * dinv_ref[...]
        out_ref[...] = _split(g2)

    return pl.pallas_call(
        body,
        grid=(n_pad // _BN,),
        in_specs=[
            pl.BlockSpec((NC, _BN, hd), lambda i: (0, i, 0)),
            pl.BlockSpec((NC, _BN, hd), lambda i: (0, i, 0)),
            pl.BlockSpec((_BN, 1), lambda i: (i, 0)),
            pl.BlockSpec((d,), lambda i: (0,)),
            pl.BlockSpec((d, d), lambda i: (0, 0)),
        ],
        out_specs=pl.BlockSpec((NC, _BN, hd), lambda i: (0, i, 0)),
        out_shape=jax.ShapeDtypeStruct((NC, n_pad, hd), jnp.float32),
    )(p, g1, dinv, b1, w2)


def _tc_last(p, g2, dinv, b2):
    _, n_pad, hd = g2.shape
    d = 2 * hd

    def body(p_ref, g_ref, dinv_ref, b_ref, out_ref):
        s = jnp.concatenate([p_ref[0] + g_ref[0], p_ref[1] + g_ref[1]],
                            axis=-1)
        out_ref[...] = dinv_ref[...] * s + b_ref[...][None, :]

    return pl.pallas_call(
        body,
        grid=(n_pad // _BN,),
        in_specs=[
            pl.BlockSpec((NC, _BN, hd), lambda i: (0, i, 0)),
            pl.BlockSpec((NC, _BN, hd), lambda i: (0, i, 0)),
            pl.BlockSpec((_BN, 1), lambda i: (i, 0)),
            pl.BlockSpec((d,), lambda i: (0,)),
        ],
        out_specs=pl.BlockSpec((_BN, d), lambda i: (i, 0)),
        out_shape=jax.ShapeDtypeStruct((n_pad, d), jnp.float32),
    )(p, g2, dinv, b2)


def kernel(x, edge_index, W1, b1, W2, b2):
    n, d = x.shape
    e = edge_index.shape[1]
    hd = d // 2

    n_pad = ((n + 1 + _BN - 1) // _BN) * _BN
    gpm = 8 * NBUF
    gps = -(-e // (NS * EG))
    gps = -(-gps // gpm) * gpm
    e_pad = NS * gps * EG

    src = edge_index[0]
    dst = edge_index[1]
    pad = e_pad - e
    srcp = jnp.concatenate([src, jnp.zeros((pad,), edge_index.dtype)])
    dstp = jnp.concatenate([dst, jnp.full((pad,), n, edge_index.dtype)])
    src2 = srcp.reshape(NS * gps, EG)
    dst2 = dstp.reshape(NS * gps, EG)
    xp = jnp.pad(x, ((0, n_pad - n), (0, 0)))
    z = jnp.zeros((n_pad // NS, hd), jnp.float32)

    deg_k = _make_deg_kernel(n_pad, e_pad)
    edge_k = _make_edge_kernel(n_pad, NS * gps, hd)

    degs = deg_k(dstp)
    g1, dinv = _tc_first(xp, W1, degs)
    p1 = edge_k(g1, src2, dst2, z)
    g2 = _tc_mid(p1, g1, dinv, b1, W2)
    p2 = edge_k(g2, src2, dst2, z)
    out = _tc_last(p2, g2, dinv, b2)
    return out[:n]


# speedup vs baseline: 1.0433x; 1.0433x over previous
import functools

import jax
import jax.numpy as jnp
from jax import lax
from jax.experimental import pallas as pl
from jax.experimental.pallas import tpu as pltpu
from jax.experimental.pallas import tpu_sc as plsc

NC = 2
NS = 16
NW = NC * NS
LANE = 16
EG = 128
NBUF = 4


def _make_deg_kernel(n_pad: int, e_pad: int):
    per_w = e_pad // NW
    mesh = plsc.VectorSubcoreMesh(
        core_axis_name="c", subcore_axis_name="s",
        num_cores=NC, num_subcores=NS)

    @functools.partial(
        pl.kernel, mesh=mesh,
        compiler_params=pltpu.CompilerParams(needs_layout_passes=False),
        out_type=jax.ShapeDtypeStruct((NW, n_pad), jnp.float32),
        scratch_types=[
            pltpu.VMEM((per_w,), jnp.int32),
            pltpu.VMEM((n_pad,), jnp.float32),
        ],
    )
    def deg_kernel(dst_hbm, out_hbm, dst_v, hist_v):
        cid = lax.axis_index("c")
        sid = lax.axis_index("s")
        wid = cid * NS + sid
        pltpu.sync_copy(dst_hbm.at[pl.ds(wid * per_w, per_w)], dst_v)
        zeros16 = jnp.zeros((LANE,), jnp.float32)
        ones16 = jnp.ones((LANE,), jnp.float32)

        def zero_body(i, carry):
            hist_v[pl.ds(i * LANE, LANE)] = zeros16
            return carry

        lax.fori_loop(0, n_pad // LANE, zero_body, 0)

        def acc_body(t, carry):
            idx = dst_v[pl.ds(t * LANE, LANE)]
            plsc.addupdate_scatter(hist_v, [idx], ones16)
            return carry

        lax.fori_loop(0, per_w // LANE, acc_body, 0)
        pltpu.sync_copy(hist_v, out_hbm.at[wid])

    return deg_kernel


def _make_edge_kernel(n_pad: int, groups_total: int, hd: int):
    gps = groups_total // NS
    zrows = n_pad // NS
    mesh = plsc.VectorSubcoreMesh(
        core_axis_name="c", subcore_axis_name="s",
        num_cores=NC, num_subcores=NS)

    @functools.partial(
        pl.kernel, mesh=mesh,
        compiler_params=pltpu.CompilerParams(
            needs_layout_passes=False, use_tc_tiling_on_sc=False),
        out_type=jax.ShapeDtypeStruct((NC, n_pad, hd), jnp.float32),
        scratch_types=[
            pltpu.VMEM((gps, EG), jnp.int32),
            pltpu.VMEM((gps, EG), jnp.int32),
        ] + [pltpu.VMEM((EG, hd), jnp.float32) for _ in range(NBUF)] + [
            pltpu.VMEM_SHARED((n_pad, hd), jnp.float32),
        ] + [pltpu.SemaphoreType.DMA for _ in range(NBUF)],
    )
    def edge_kernel(g_hbm, src_hbm, dst_hbm, z_hbm, out_hbm,
                    src_v, dst_v, *rest):
        rows = rest[:NBUF]
        acc = rest[NBUF]
        sems = rest[NBUF + 1:]
        cid = lax.axis_index("c")
        sid = lax.axis_index("s")
        table = g_hbm.at[cid]
        pltpu.sync_copy(z_hbm, acc.at[pl.ds(sid * zrows, zrows)])
        pltpu.sync_copy(src_hbm.at[pl.ds(sid * gps, gps)], src_v)
        pltpu.sync_copy(dst_hbm.at[pl.ds(sid * gps, gps)], dst_v)
        for b in range(NBUF):
            pltpu.async_copy(table.at[src_v.at[b]], rows[b], sems[b])
        plsc.subcore_barrier()

        def chunk_body(jc, carry):
            j0 = jc * NBUF
            for b in range(NBUF):
                j = j0 + b
                pltpu.make_async_copy(
                    table.at[src_v.at[j]], rows[b], sems[b]).wait()
                pltpu.sync_copy(rows[b], acc.at[dst_v.at[j]], add=True)
                pltpu.async_copy(
                    table.at[src_v.at[j + NBUF]], rows[b], sems[b])
            return carry

        lax.fori_loop(0, gps // NBUF - 1, chunk_body, 0)
        for b in range(NBUF):
            j = gps - NBUF + b
            pltpu.make_async_copy(
                table.at[src_v.at[j]], rows[b], sems[b]).wait()
            pltpu.sync_copy(rows[b], acc.at[dst_v.at[j]], add=True)
        plsc.subcore_barrier()
        pltpu.sync_copy(acc.at[pl.ds(sid * zrows, zrows)],
                        out_hbm.at[cid, pl.ds(sid * zrows, zrows)])

    return edge_kernel


_BN = 1024


def _split(h):
    hd = h.shape[-1] // 2
    return jnp.stack([h[:, :hd], h[:, hd:]], axis=0)


def _tc_first(x, w1, degs):
    n_pad, d = x.shape

    def body(x_ref, w_ref, deg_ref, g_ref, dinv_ref):
        deg = jnp.sum(deg_ref[...], axis=0) + 1.0
        dinv = lax.rsqrt(deg)[:, None]
        h = jnp.dot(x_ref[...], w_ref[...],
                    preferred_element_type=jnp.float32)
        g_ref[...] = _split(h * dinv)
        dinv_ref[...] = dinv

    return pl.pallas_call(
        body,
        grid=(n_pad // _BN,),
        in_specs=[
            pl.BlockSpec((_BN, d), lambda i: (i, 0)),
            pl.BlockSpec((d, d), lambda i: (0, 0)),
            pl.BlockSpec((NW, _BN), lambda i: (0, i)),
        ],
        out_specs=[
            pl.BlockSpec((NC, _BN, d // 2), lambda i: (0, i, 0)),
            pl.BlockSpec((_BN, 1), lambda i: (i, 0)),
        ],
        out_shape=[
            jax.ShapeDtypeStruct((NC, n_pad, d // 2), jnp.float32),
            jax.ShapeDtypeStruct((n_pad, 1), jnp.float32),
        ],
    )(x, w1, degs)


def _tc_mid(p, g1, dinv, b1, w2):
    _, n_pad, hd = g1.shape
    d = 2 * hd

    def body(p_ref, g_ref, dinv_ref, b_ref, w_ref, out_ref):
        s = jnp.concatenate([p_ref[0] + g_ref[0], p_ref[1] + g_ref[1]],
                            axis=-1)
        h = jnp.maximum(dinv_ref[...] * s + b_ref[...][None, :], 0.0)
        g2 = jnp.dot(h, w_ref[...],
                     preferred_element_type=jnp.float32) * dinv_ref[...]
        out_ref[...] = _split(g2)

    return pl.pallas_call(
        body,
        grid=(n_pad // _BN,),
        in_specs=[
            pl.BlockSpec((NC, _BN, hd), lambda i: (0, i, 0)),
            pl.BlockSpec((NC, _BN, hd), lambda i: (0, i, 0)),
            pl.BlockSpec((_BN, 1), lambda i: (i, 0)),
            pl.BlockSpec((d,), lambda i: (0,)),
            pl.BlockSpec((d, d), lambda i: (0, 0)),
        ],
        out_specs=pl.BlockSpec((NC, _BN, hd), lambda i: (0, i, 0)),
        out_shape=jax.ShapeDtypeStruct((NC, n_pad, hd), jnp.float32),
    )(p, g1, dinv, b1, w2)


def _tc_last(p, g2, dinv, b2):
    _, n_pad, hd = g2.shape
    d = 2 * hd

    def body(p_ref, g_ref, dinv_ref, b_ref, out_ref):
        s = jnp.concatenate([p_ref[0] + g_ref[0], p_ref[1] + g_ref[1]],
                            axis=-1)
        out_ref[...] = dinv_ref[...] * s + b_ref[...][None, :]

    return pl.pallas_call(
        body,
        grid=(n_pad // _BN,),
        in_specs=[
            pl.BlockSpec((NC, _BN, hd), lambda i: (0, i, 0)),
            pl.BlockSpec((NC, _BN, hd), lambda i: (0, i, 0)),
            pl.BlockSpec((_BN, 1), lambda i: (i, 0)),
            pl.BlockSpec((d,), lambda i: (0,)),
        ],
        out_specs=pl.BlockSpec((_BN, d), lambda i: (i, 0)),
        out_shape=jax.ShapeDtypeStruct((n_pad, d), jnp.float32),
    )(p, g2, dinv, b2)


def kernel(x, edge_index, W1, b1, W2, b2):
    n, d = x.shape
    e = edge_index.shape[1]
    hd = d // 2

    n_pad = ((n + 1 + _BN - 1) // _BN) * _BN
    gpm = 8 * NBUF
    gps = -(-e // (NS * EG))
    gps = -(-gps // gpm) * gpm
    e_pad = NS * gps * EG

    src = edge_index[0]
    dst = edge_index[1]
    pad = e_pad - e
    srcp = jnp.concatenate([src, jnp.zeros((pad,), edge_index.dtype)])
    dstp = jnp.concatenate([dst, jnp.full((pad,), n, edge_index.dtype)])
    src2 = srcp.reshape(NS * gps, EG)
    dst2 = dstp.reshape(NS * gps, EG)
    xp = jnp.pad(x, ((0, n_pad - n), (0, 0)))
    z = jnp.zeros((n_pad // NS, hd), jnp.float32)

    deg_k = _make_deg_kernel(n_pad, e_pad)
    edge_k = _make_edge_kernel(n_pad, NS * gps, hd)

    degs = deg_k(dstp)
    g1, dinv = _tc_first(xp, W1, degs)
    p1 = edge_k(g1, src2, dst2, z)
    g2 = _tc_mid(p1, g1, dinv, b1, W2)
    p2 = edge_k(g2, src2, dst2, z)
    out = _tc_last(p2, g2, dinv, b2)
    return out[:n]
